# Initial kernel scaffold; baseline (speedup 1.0000x reference)
import jax
import jax.numpy as jnp
from jax.experimental import pallas as pl


def kernel(x, edge_index, edge_weight, W0, b0, W1, b1):
    raise NotImplementedError("write your pallas kernel here")


# speedup vs baseline: 3.8159x; 3.8159x over previous
import functools

import jax
import jax.numpy as jnp
from jax import lax
from jax.experimental import pallas as pl
from jax.experimental.pallas import tpu as pltpu
from jax.experimental.pallas import tpu_sc as plsc

NC, NS, LANES = 2, 16, 16
NW = NC * NS


def _mm0_body(x_ref, w_ref, o_ref):
    o_ref[...] = jnp.dot(x_ref[...], w_ref[...],
                         preferred_element_type=jnp.float32)


def _mm1_body(p_ref, b_ref, w_ref, o_ref):
    h = p_ref[0] + p_ref[1] + b_ref[...]
    h = jnp.maximum(h, 0.0)
    o_ref[...] = jnp.dot(h, w_ref[...], preferred_element_type=jnp.float32)


def _merge_body(p_ref, b_ref, o_ref):
    o_ref[...] = p_ref[0] + p_ref[1] + b_ref[...]


def _mm0(x, w, rows_per_block):
    n, k = x.shape
    m = w.shape[1]
    return pl.pallas_call(
        _mm0_body,
        grid=(n // rows_per_block,),
        in_specs=[
            pl.BlockSpec((rows_per_block, k), lambda i: (i, 0)),
            pl.BlockSpec((k, m), lambda i: (0, 0)),
        ],
        out_specs=pl.BlockSpec((rows_per_block, m), lambda i: (i, 0)),
        out_shape=jax.ShapeDtypeStruct((n, m), jnp.float32),
    )(x, w)


def _mm1(p, b, w, rows_per_block):
    _, n, k = p.shape
    m = w.shape[1]
    return pl.pallas_call(
        _mm1_body,
        grid=(n // rows_per_block,),
        in_specs=[
            pl.BlockSpec((2, rows_per_block, k), lambda i: (0, i, 0)),
            pl.BlockSpec((1, k), lambda i: (0, 0)),
            pl.BlockSpec((k, m), lambda i: (0, 0)),
        ],
        out_specs=pl.BlockSpec((rows_per_block, m), lambda i: (i, 0)),
        out_shape=jax.ShapeDtypeStruct((n, m), jnp.float32),
    )(p, b, w)


def _merge(p, b, rows_per_block):
    _, n, k = p.shape
    return pl.pallas_call(
        _merge_body,
        grid=(n // rows_per_block,),
        in_specs=[
            pl.BlockSpec((2, rows_per_block, k), lambda i: (0, i, 0)),
            pl.BlockSpec((1, k), lambda i: (0, 0)),
        ],
        out_specs=pl.BlockSpec((rows_per_block, k), lambda i: (i, 0)),
        out_shape=jax.ShapeDtypeStruct((n, k), jnp.float32),
    )(p, b)


def _make_spmm(n_nodes, d, n_edges):
    assert n_edges % NW == 0
    epw = n_edges // NW
    chunk = 80
    assert epw % chunk == 0
    n_chunks = epw // chunk
    n_groups = chunk // LANES
    rpt = (n_nodes // NS) // 8 * 8
    tail = n_nodes - NS * rpt
    assert 0 <= tail <= rpt and tail % 8 == 0
    dsub = d // LANES

    mesh = plsc.VectorSubcoreMesh(
        core_axis_name="c", subcore_axis_name="s",
        num_cores=NC, num_subcores=NS)

    @functools.partial(
        pl.kernel,
        out_type=jax.ShapeDtypeStruct((NC, n_nodes, d), jnp.float32),
        mesh=mesh,
        compiler_params=pltpu.CompilerParams(needs_layout_passes=False),
        scratch_types=[
            pltpu.VMEM_SHARED((n_nodes, d), jnp.float32),
            pltpu.VMEM((chunk,), jnp.int32),
            pltpu.VMEM((chunk,), jnp.int32),
            pltpu.VMEM((chunk,), jnp.float32),
            pltpu.VMEM((chunk, d), jnp.float32),
            pltpu.SemaphoreType.DMA,
        ],
    )
    def spmm(h_hbm, src_hbm, dst_hbm, w_hbm, out_hbm,
             acc, src_v, dst_v, w_v, rows_v, sem):
        cid = lax.axis_index("c")
        sid = lax.axis_index("s")
        ebase = (cid * NS + sid) * epw
        rb = sid * rpt

        def zrow(r, carry):
            for k in range(dsub):
                rows_v[r, pl.ds(k * LANES, LANES)] = jnp.zeros(
                    (LANES,), jnp.float32)
            return carry
        lax.fori_loop(0, chunk, zrow, 0)

        def zcopy(j, carry):
            pltpu.sync_copy(rows_v, acc.at[pl.ds(rb + j * chunk, chunk)])
            return carry
        nzc = rpt // chunk
        lax.fori_loop(0, nzc, zcopy, 0)
        zrem = rpt - nzc * chunk
        if zrem:
            pltpu.sync_copy(rows_v.at[pl.ds(0, zrem)],
                            acc.at[pl.ds(rb + nzc * chunk, zrem)])
        if tail:
            @pl.when(sid == NS - 1)
            def _zero_tail():
                pltpu.sync_copy(rows_v.at[pl.ds(0, tail)],
                                acc.at[pl.ds(NS * rpt, tail)])
        plsc.subcore_barrier()

        def do_chunk(ci, carry):
            e0 = ebase + ci * chunk
            pltpu.sync_copy(src_hbm.at[pl.ds(e0, chunk)], src_v)
            pltpu.sync_copy(dst_hbm.at[pl.ds(e0, chunk)], dst_v)
            pltpu.sync_copy(w_hbm.at[pl.ds(e0, chunk)], w_v)
            pltpu.async_copy(h_hbm.at[src_v], rows_v, sem).wait()

            def group(g, c2):
                b16 = g * LANES
                for i in range(LANES):
                    e = b16 + i
                    wi = plsc.load_gather(
                        w_v, [jnp.full((LANES,), e, jnp.int32)])
                    for k in range(dsub):
                        sl = pl.ds(k * LANES, LANES)
                        rows_v[e, sl] = rows_v[e, sl] * wi
                return c2
            lax.fori_loop(0, n_groups, group, 0)
            pltpu.sync_copy(rows_v, acc.at[dst_v], add=True)
            return carry
        lax.fori_loop(0, n_chunks, do_chunk, 0)

        plsc.subcore_barrier()
        pltpu.sync_copy(acc.at[pl.ds(rb, rpt)],
                        out_hbm.at[cid, pl.ds(rb, rpt)])
        if tail:
            @pl.when(sid == NS - 1)
            def _write_tail():
                pltpu.sync_copy(acc.at[pl.ds(NS * rpt, tail)],
                                out_hbm.at[cid, pl.ds(NS * rpt, tail)])

    return spmm


def kernel(x, edge_index, edge_weight, W0, b0, W1, b1):
    n, _ = x.shape
    e = edge_index.shape[1]
    src = edge_index[0].astype(jnp.int32)
    dst = edge_index[1].astype(jnp.int32)
    ew = edge_weight.astype(jnp.float32)

    spmm = _make_spmm(n, W0.shape[1], e)

    h0 = _mm0(x, W0, 1000)
    p0 = spmm(h0, src, dst, ew)
    h1 = _mm1(p0, b0.reshape(1, -1), W1, 1000)
    p1 = spmm(h1, src, dst, ew)
    return _merge(p1, b1.reshape(1, -1), 1000)


# speedup vs baseline: 10.8226x; 2.8362x over previous
import functools

import jax
import jax.numpy as jnp
from jax import lax
from jax.experimental import pallas as pl
from jax.experimental.pallas import tpu as pltpu
from jax.experimental.pallas import tpu_sc as plsc

NC, NS, LANES = 2, 16, 16
NW = NC * NS


def _mm0_body(x_ref, w_ref, o_ref):
    o_ref[...] = jnp.dot(x_ref[...], w_ref[...],
                         preferred_element_type=jnp.float32)


def _mm1_body(p_ref, b_ref, w_ref, o_ref):
    h = p_ref[0] + p_ref[1] + b_ref[...]
    h = jnp.maximum(h, 0.0)
    o_ref[...] = jnp.dot(h, w_ref[...], preferred_element_type=jnp.float32)


def _merge_body(p_ref, b_ref, o_ref):
    o_ref[...] = p_ref[0] + p_ref[1] + b_ref[...]


def _mm0(x, w, rows_per_block):
    n, k = x.shape
    m = w.shape[1]
    return pl.pallas_call(
        _mm0_body,
        grid=(n // rows_per_block,),
        in_specs=[
            pl.BlockSpec((rows_per_block, k), lambda i: (i, 0)),
            pl.BlockSpec((k, m), lambda i: (0, 0)),
        ],
        out_specs=pl.BlockSpec((rows_per_block, m), lambda i: (i, 0)),
        out_shape=jax.ShapeDtypeStruct((n, m), jnp.float32),
    )(x, w)


def _mm1(p, b, w, rows_per_block):
    _, n, k = p.shape
    m = w.shape[1]
    return pl.pallas_call(
        _mm1_body,
        grid=(n // rows_per_block,),
        in_specs=[
            pl.BlockSpec((2, rows_per_block, k), lambda i: (0, i, 0)),
            pl.BlockSpec((1, k), lambda i: (0, 0)),
            pl.BlockSpec((k, m), lambda i: (0, 0)),
        ],
        out_specs=pl.BlockSpec((rows_per_block, m), lambda i: (i, 0)),
        out_shape=jax.ShapeDtypeStruct((n, m), jnp.float32),
    )(p, b, w)


def _merge(p, b, rows_per_block):
    _, n, k = p.shape
    return pl.pallas_call(
        _merge_body,
        grid=(n // rows_per_block,),
        in_specs=[
            pl.BlockSpec((2, rows_per_block, k), lambda i: (0, i, 0)),
            pl.BlockSpec((1, k), lambda i: (0, 0)),
        ],
        out_specs=pl.BlockSpec((rows_per_block, k), lambda i: (i, 0)),
        out_shape=jax.ShapeDtypeStruct((n, k), jnp.float32),
    )(p, b)


def _make_spmm(n_nodes, d, n_edges):
    assert n_edges % NW == 0
    epw = n_edges // NW
    chunk = 80
    assert epw % chunk == 0
    n_chunks = epw // chunk
    n_groups = chunk // LANES
    rpt = (n_nodes // NS) // 8 * 8
    tail = n_nodes - NS * rpt
    assert 0 <= tail <= rpt and tail % 8 == 0
    dsub = d // LANES

    mesh = plsc.VectorSubcoreMesh(
        core_axis_name="c", subcore_axis_name="s",
        num_cores=NC, num_subcores=NS)

    NBUF = 3

    @functools.partial(
        pl.kernel,
        out_type=jax.ShapeDtypeStruct((NC, n_nodes, d), jnp.float32),
        mesh=mesh,
        compiler_params=pltpu.CompilerParams(needs_layout_passes=False),
        scratch_types=[
            pltpu.VMEM_SHARED((n_nodes, d), jnp.float32),
            pltpu.VMEM((epw,), jnp.int32),
            [pltpu.VMEM((chunk,), jnp.int32)] * NBUF,
            [pltpu.VMEM((chunk,), jnp.float32)] * NBUF,
            [pltpu.VMEM((chunk, d), jnp.float32)] * NBUF,
            [pltpu.SemaphoreType.DMA] * NBUF,
            [pltpu.SemaphoreType.DMA] * NBUF,
            [pltpu.SemaphoreType.DMA] * NBUF,
        ],
    )
    def spmm(h_hbm, src_hbm, dst_hbm, w_hbm, out_hbm,
             acc, src_all, dst_v, w_v, rows, gsem, lsem, ssem):
        cid = lax.axis_index("c")
        sid = lax.axis_index("s")
        ebase = (cid * NS + sid) * epw
        rb = sid * rpt

        pltpu.sync_copy(src_hbm.at[pl.ds(ebase, epw)], src_all)

        def zrow(r, carry):
            for k in range(dsub):
                rows[0][r, pl.ds(k * LANES, LANES)] = jnp.zeros(
                    (LANES,), jnp.float32)
            return carry
        lax.fori_loop(0, chunk, zrow, 0)

        def zcopy(j, carry):
            pltpu.sync_copy(rows[0], acc.at[pl.ds(rb + j * chunk, chunk)])
            return carry
        nzc = rpt // chunk
        lax.fori_loop(0, nzc, zcopy, 0)
        zrem = rpt - nzc * chunk
        if zrem:
            pltpu.sync_copy(rows[0].at[pl.ds(0, zrem)],
                            acc.at[pl.ds(rb + nzc * chunk, zrem)])
        if tail:
            @pl.when(sid == NS - 1)
            def _zero_tail():
                pltpu.sync_copy(rows[0].at[pl.ds(0, tail)],
                                acc.at[pl.ds(NS * rpt, tail)])
        plsc.subcore_barrier()

        def issue_chunk(k, b):
            e0 = ebase + k * chunk
            pltpu.async_copy(dst_hbm.at[pl.ds(e0, chunk)], dst_v[b],
                             lsem[b])
            pltpu.async_copy(w_hbm.at[pl.ds(e0, chunk)], w_v[b], lsem[b])
            pltpu.async_copy(h_hbm.at[src_all.at[pl.ds(k * chunk, chunk)]],
                             rows[b], gsem[b])

        def wait_loads(b):
            pltpu.make_async_copy(dst_hbm.at[pl.ds(0, chunk)], dst_v[b],
                                  lsem[b]).wait()
            pltpu.make_async_copy(w_hbm.at[pl.ds(0, chunk)], w_v[b],
                                  lsem[b]).wait()

        def wait_gather(b):
            pltpu.make_async_copy(
                h_hbm.at[src_all.at[pl.ds(0, chunk)]], rows[b],
                gsem[b]).wait()

        def wait_scatter(b):
            pltpu.make_async_copy(rows[b], acc.at[dst_v[b]],
                                  ssem[b]).wait()

        def scale(b):
            def group(g, c2):
                b16 = g * LANES
                for i in range(LANES):
                    e = b16 + i
                    wi = plsc.load_gather(
                        w_v[b], [jnp.full((LANES,), e, jnp.int32)])
                    for k in range(dsub):
                        sl = pl.ds(k * LANES, LANES)
                        rows[b][e, sl] = rows[b][e, sl] * wi
                return c2
            lax.fori_loop(0, n_groups, group, 0)

        def body(k, b, issue_next, guard_scatter):
            bn = (b + 1) % NBUF
            if issue_next:
                if guard_scatter:
                    @pl.when(k >= 2)
                    def _w():
                        wait_scatter(bn)
                issue_chunk(k + 1, bn)
            wait_gather(b)
            wait_loads(b)
            scale(b)
            pltpu.async_copy(rows[b], acc.at[dst_v[b]], ssem[b], add=True)

        issue_chunk(0, 0)

        n_main = (n_chunks - 2) // NBUF
        def triple(g, carry):
            k0 = g * NBUF
            for j in range(NBUF):
                body(k0 + j, j, True, True)
            return carry
        lax.fori_loop(0, n_main, triple, 0)

        for k in range(n_main * NBUF, n_chunks):
            body(k, k % NBUF, k + 1 < n_chunks, True)
        for b in range(NBUF):
            wait_scatter(b)

        plsc.subcore_barrier()
        pltpu.sync_copy(acc.at[pl.ds(rb, rpt)],
                        out_hbm.at[cid, pl.ds(rb, rpt)])
        if tail:
            @pl.when(sid == NS - 1)
            def _write_tail():
                pltpu.sync_copy(acc.at[pl.ds(NS * rpt, tail)],
                                out_hbm.at[cid, pl.ds(NS * rpt, tail)])

    return spmm


def kernel(x, edge_index, edge_weight, W0, b0, W1, b1):
    n, _ = x.shape
    e = edge_index.shape[1]
    src = edge_index[0].astype(jnp.int32)
    dst = edge_index[1].astype(jnp.int32)
    ew = edge_weight.astype(jnp.float32)

    spmm = _make_spmm(n, W0.shape[1], e)

    h0 = _mm0(x, W0, 1000)
    p0 = spmm(h0, src, dst, ew)
    h1 = _mm1(p0, b0.reshape(1, -1), W1, 1000)
    p1 = spmm(h1, src, dst, ew)
    return _merge(p1, b1.reshape(1, -1), 1000)


# speedup vs baseline: 11.5781x; 1.0698x over previous
import functools

import jax
import jax.numpy as jnp
from jax import lax
from jax.experimental import pallas as pl
from jax.experimental.pallas import tpu as pltpu
from jax.experimental.pallas import tpu_sc as plsc

NC, NS, LANES = 2, 16, 16
NW = NC * NS


def _mm0_body(x_ref, w_ref, o_ref):
    o_ref[...] = jnp.dot(x_ref[...], w_ref[...],
                         preferred_element_type=jnp.float32)


def _mm1_body(p_ref, b_ref, w_ref, o_ref):
    h = p_ref[0] + p_ref[1] + b_ref[...]
    h = jnp.maximum(h, 0.0)
    o_ref[...] = jnp.dot(h, w_ref[...], preferred_element_type=jnp.float32)


def _merge_body(p_ref, b_ref, o_ref):
    o_ref[...] = p_ref[0] + p_ref[1] + b_ref[...]


def _mm0(x, w, rows_per_block):
    n, k = x.shape
    m = w.shape[1]
    return pl.pallas_call(
        _mm0_body,
        grid=(n // rows_per_block,),
        in_specs=[
            pl.BlockSpec((rows_per_block, k), lambda i: (i, 0)),
            pl.BlockSpec((k, m), lambda i: (0, 0)),
        ],
        out_specs=pl.BlockSpec((rows_per_block, m), lambda i: (i, 0)),
        out_shape=jax.ShapeDtypeStruct((n, m), jnp.float32),
    )(x, w)


def _mm1(p, b, w, rows_per_block):
    _, n, k = p.shape
    m = w.shape[1]
    return pl.pallas_call(
        _mm1_body,
        grid=(n // rows_per_block,),
        in_specs=[
            pl.BlockSpec((2, rows_per_block, k), lambda i: (0, i, 0)),
            pl.BlockSpec((1, k), lambda i: (0, 0)),
            pl.BlockSpec((k, m), lambda i: (0, 0)),
        ],
        out_specs=pl.BlockSpec((rows_per_block, m), lambda i: (i, 0)),
        out_shape=jax.ShapeDtypeStruct((n, m), jnp.float32),
    )(p, b, w)


def _merge(p, b, rows_per_block):
    _, n, k = p.shape
    return pl.pallas_call(
        _merge_body,
        grid=(n // rows_per_block,),
        in_specs=[
            pl.BlockSpec((2, rows_per_block, k), lambda i: (0, i, 0)),
            pl.BlockSpec((1, k), lambda i: (0, 0)),
        ],
        out_specs=pl.BlockSpec((rows_per_block, k), lambda i: (i, 0)),
        out_shape=jax.ShapeDtypeStruct((n, k), jnp.float32),
    )(p, b)


def _make_spmm(n_nodes, d, n_edges):
    assert n_edges % NW == 0
    epw = n_edges // NW
    chunk = 80
    assert epw % chunk == 0
    n_chunks = epw // chunk
    n_groups = chunk // LANES
    rpt = (n_nodes // NS) // 8 * 8
    tail = n_nodes - NS * rpt
    assert 0 <= tail <= rpt and tail % 8 == 0
    dsub = d // LANES

    mesh = plsc.VectorSubcoreMesh(
        core_axis_name="c", subcore_axis_name="s",
        num_cores=NC, num_subcores=NS)

    NBUF = 4
    LOOK = 2
    SBUF = 8
    SL = 4

    @functools.partial(
        pl.kernel,
        out_type=jax.ShapeDtypeStruct((NC, n_nodes, d), jnp.float32),
        mesh=mesh,
        compiler_params=pltpu.CompilerParams(needs_layout_passes=False),
        scratch_types=[
            pltpu.VMEM_SHARED((n_nodes, d), jnp.float32),
            [pltpu.VMEM((chunk,), jnp.int32)] * SBUF,
            [pltpu.VMEM((chunk,), jnp.int32)] * NBUF,
            [pltpu.VMEM((chunk,), jnp.float32)] * NBUF,
            [pltpu.VMEM((chunk, d), jnp.float32)] * NBUF,
            [pltpu.SemaphoreType.DMA] * SBUF,
            [pltpu.SemaphoreType.DMA] * NBUF,
            [pltpu.SemaphoreType.DMA] * NBUF,
            [pltpu.SemaphoreType.DMA] * NBUF,
        ],
    )
    def spmm(h_hbm, src_hbm, dst_hbm, w_hbm, out_hbm,
             acc, sv, dst_v, w_v, rows, ksem, gsem, lsem, ssem):
        cid = lax.axis_index("c")
        sid = lax.axis_index("s")
        ebase = (cid * NS + sid) * epw
        rb = sid * rpt

        def zrow(r, carry):
            for k in range(dsub):
                rows[0][r, pl.ds(k * LANES, LANES)] = jnp.zeros(
                    (LANES,), jnp.float32)
            return carry
        lax.fori_loop(0, chunk, zrow, 0)

        def zcopy(j, carry):
            pltpu.sync_copy(rows[0], acc.at[pl.ds(rb + j * chunk, chunk)])
            return carry
        nzc = rpt // chunk
        lax.fori_loop(0, nzc, zcopy, 0)
        zrem = rpt - nzc * chunk
        if zrem:
            pltpu.sync_copy(rows[0].at[pl.ds(0, zrem)],
                            acc.at[pl.ds(rb + nzc * chunk, zrem)])
        if tail:
            @pl.when(sid == NS - 1)
            def _zero_tail():
                pltpu.sync_copy(rows[0].at[pl.ds(0, tail)],
                                acc.at[pl.ds(NS * rpt, tail)])
        plsc.subcore_barrier()

        def issue_src(k, bs):
            pltpu.async_copy(src_hbm.at[pl.ds(ebase + k * chunk, chunk)],
                             sv[bs], ksem[bs])

        def wait_src(bs):
            pltpu.make_async_copy(src_hbm.at[pl.ds(0, chunk)], sv[bs],
                                  ksem[bs]).wait()

        def issue_chunk(k, sn, b):
            e0 = ebase + k * chunk
            pltpu.async_copy(dst_hbm.at[pl.ds(e0, chunk)], dst_v[b],
                             lsem[b])
            pltpu.async_copy(w_hbm.at[pl.ds(e0, chunk)], w_v[b], lsem[b])
            pltpu.async_copy(h_hbm.at[sv[sn]], rows[b], gsem[b])

        def wait_loads(b):
            pltpu.make_async_copy(dst_hbm.at[pl.ds(0, chunk)], dst_v[b],
                                  lsem[b]).wait()
            pltpu.make_async_copy(w_hbm.at[pl.ds(0, chunk)], w_v[b],
                                  lsem[b]).wait()

        def wait_gather(sn, b):
            pltpu.make_async_copy(h_hbm.at[sv[sn]], rows[b],
                                  gsem[b]).wait()

        def wait_scatter(b):
            pltpu.make_async_copy(rows[b], acc.at[dst_v[b]],
                                  ssem[b]).wait()

        def scale(b):
            def group(g, c2):
                b16 = g * LANES
                for i in range(LANES):
                    e = b16 + i
                    wi = plsc.load_gather(
                        w_v[b], [jnp.full((LANES,), e, jnp.int32)])
                    for k in range(dsub):
                        sl = pl.ds(k * LANES, LANES)
                        rows[b][e, sl] = rows[b][e, sl] * wi
                return c2
            lax.fori_loop(0, n_groups, group, 0)

        def body(k, j, issue_next, issue_src_next):
            b = j % NBUF
            if issue_src_next:
                issue_src(k + SL, (j + SL) % SBUF)
            if issue_next:
                bn = (j + LOOK) % NBUF
                sn = (j + LOOK) % SBUF
                @pl.when(k >= NBUF - LOOK)
                def _w():
                    wait_scatter(bn)
                wait_src(sn)
                issue_chunk(k + LOOK, sn, bn)
            wait_gather(j % SBUF, b)
            wait_loads(b)
            scale(b)
            pltpu.async_copy(rows[b], acc.at[dst_v[b]], ssem[b], add=True)

        for j in range(SL):
            issue_src(j, j % SBUF)
        for j in range(LOOK):
            wait_src(j % SBUF)
            issue_chunk(j, j % SBUF, j % NBUF)

        UNROLL = 8
        n_main = (n_chunks - SL) // UNROLL
        def group(g, carry):
            k0 = g * UNROLL
            for j in range(UNROLL):
                body(k0 + j, j, True, True)
            return carry
        lax.fori_loop(0, n_main, group, 0)

        for k in range(n_main * UNROLL, n_chunks):
            body(k, k % UNROLL, k + LOOK < n_chunks, k + SL < n_chunks)
        for b in range(NBUF):
            wait_scatter(b)

        plsc.subcore_barrier()
        pltpu.sync_copy(acc.at[pl.ds(rb, rpt)],
                        out_hbm.at[cid, pl.ds(rb, rpt)])
        if tail:
            @pl.when(sid == NS - 1)
            def _write_tail():
                pltpu.sync_copy(acc.at[pl.ds(NS * rpt, tail)],
                                out_hbm.at[cid, pl.ds(NS * rpt, tail)])

    return spmm


def kernel(x, edge_index, edge_weight, W0, b0, W1, b1):
    n, _ = x.shape
    e = edge_index.shape[1]
    src = edge_index[0].astype(jnp.int32)
    dst = edge_index[1].astype(jnp.int32)
    ew = edge_weight.astype(jnp.float32)

    spmm = _make_spmm(n, W0.shape[1], e)

    h0 = _mm0(x, W0, 1000)
    p0 = spmm(h0, src, dst, ew)
    h1 = _mm1(p0, b0.reshape(1, -1), W1, 1000)
    p1 = spmm(h1, src, dst, ew)
    return _merge(p1, b1.reshape(1, -1), 1000)


# speedup vs baseline: 11.6428x; 1.0056x over previous
import functools

import jax
import jax.numpy as jnp
from jax import lax
from jax.experimental import pallas as pl
from jax.experimental.pallas import tpu as pltpu
from jax.experimental.pallas import tpu_sc as plsc

NC, NS, LANES = 2, 16, 16
NW = NC * NS


def _mm0_body(x_ref, w_ref, o_ref):
    o_ref[...] = jnp.dot(x_ref[...], w_ref[...],
                         preferred_element_type=jnp.float32)


def _mm1_body(p_ref, b_ref, w_ref, o_ref):
    h = p_ref[0] + p_ref[1] + b_ref[...]
    h = jnp.maximum(h, 0.0)
    o_ref[...] = jnp.dot(h, w_ref[...], preferred_element_type=jnp.float32)


def _merge_body(p_ref, b_ref, o_ref):
    o_ref[...] = p_ref[0] + p_ref[1] + b_ref[...]


def _mm0(x, w, rows_per_block):
    n, k = x.shape
    m = w.shape[1]
    return pl.pallas_call(
        _mm0_body,
        grid=(n // rows_per_block,),
        in_specs=[
            pl.BlockSpec((rows_per_block, k), lambda i: (i, 0)),
            pl.BlockSpec((k, m), lambda i: (0, 0)),
        ],
        out_specs=pl.BlockSpec((rows_per_block, m), lambda i: (i, 0)),
        out_shape=jax.ShapeDtypeStruct((n, m), jnp.float32),
    )(x, w)


def _mm1(p, b, w, rows_per_block):
    _, n, k = p.shape
    m = w.shape[1]
    return pl.pallas_call(
        _mm1_body,
        grid=(n // rows_per_block,),
        in_specs=[
            pl.BlockSpec((2, rows_per_block, k), lambda i: (0, i, 0)),
            pl.BlockSpec((1, k), lambda i: (0, 0)),
            pl.BlockSpec((k, m), lambda i: (0, 0)),
        ],
        out_specs=pl.BlockSpec((rows_per_block, m), lambda i: (i, 0)),
        out_shape=jax.ShapeDtypeStruct((n, m), jnp.float32),
    )(p, b, w)


def _merge(p, b, rows_per_block):
    _, n, k = p.shape
    return pl.pallas_call(
        _merge_body,
        grid=(n // rows_per_block,),
        in_specs=[
            pl.BlockSpec((2, rows_per_block, k), lambda i: (0, i, 0)),
            pl.BlockSpec((1, k), lambda i: (0, 0)),
        ],
        out_specs=pl.BlockSpec((rows_per_block, k), lambda i: (i, 0)),
        out_shape=jax.ShapeDtypeStruct((n, k), jnp.float32),
    )(p, b)


def _make_spmm(n_nodes, d, n_edges):
    assert n_edges % NW == 0
    epw = n_edges // NW
    chunk = 80
    assert epw % chunk == 0
    n_chunks = epw // chunk
    n_groups = chunk // LANES
    rpt = (n_nodes // NS) // 8 * 8
    tail = n_nodes - NS * rpt
    assert 0 <= tail <= rpt and tail % 8 == 0
    dsub = d // LANES

    mesh = plsc.VectorSubcoreMesh(
        core_axis_name="c", subcore_axis_name="s",
        num_cores=NC, num_subcores=NS)

    NBUF = 4
    LOOK = 2
    SBUF = 8
    SL = 4

    @functools.partial(
        pl.kernel,
        out_type=jax.ShapeDtypeStruct((NC, n_nodes, d), jnp.float32),
        mesh=mesh,
        compiler_params=pltpu.CompilerParams(needs_layout_passes=False),
        scratch_types=[
            pltpu.VMEM_SHARED((n_nodes, d), jnp.float32),
            [pltpu.VMEM((chunk,), jnp.int32)] * SBUF,
            [pltpu.VMEM((chunk,), jnp.int32)] * NBUF,
            [pltpu.VMEM((chunk,), jnp.float32)] * NBUF,
            [pltpu.VMEM((chunk, d), jnp.float32)] * NBUF,
            [pltpu.SemaphoreType.DMA] * SBUF,
            [pltpu.SemaphoreType.DMA] * NBUF,
            [pltpu.SemaphoreType.DMA] * NBUF,
            [pltpu.SemaphoreType.DMA] * NBUF,
        ],
    )
    def spmm(h_hbm, src_hbm, dst_hbm, w_hbm, out_hbm,
             acc, sv, dst_v, w_v, rows, ksem, gsem, lsem, ssem):
        cid = lax.axis_index("c")
        sid = lax.axis_index("s")
        ebase = (cid * NS + sid) * epw
        rb = sid * rpt

        def zrow(r, carry):
            for k in range(dsub):
                rows[0][r, pl.ds(k * LANES, LANES)] = jnp.zeros(
                    (LANES,), jnp.float32)
            return carry
        lax.fori_loop(0, chunk, zrow, 0)

        def zcopy(j, carry):
            pltpu.sync_copy(rows[0], acc.at[pl.ds(rb + j * chunk, chunk)])
            return carry
        nzc = rpt // chunk
        lax.fori_loop(0, nzc, zcopy, 0)
        zrem = rpt - nzc * chunk
        if zrem:
            pltpu.sync_copy(rows[0].at[pl.ds(0, zrem)],
                            acc.at[pl.ds(rb + nzc * chunk, zrem)])
        if tail:
            @pl.when(sid == NS - 1)
            def _zero_tail():
                pltpu.sync_copy(rows[0].at[pl.ds(0, tail)],
                                acc.at[pl.ds(NS * rpt, tail)])
        plsc.subcore_barrier()

        def issue_src(k, bs):
            pltpu.async_copy(src_hbm.at[pl.ds(ebase + k * chunk, chunk)],
                             sv[bs], ksem[bs])

        def wait_src(bs):
            pltpu.make_async_copy(src_hbm.at[pl.ds(0, chunk)], sv[bs],
                                  ksem[bs]).wait()

        def issue_chunk(k, sn, b):
            e0 = ebase + k * chunk
            pltpu.async_copy(dst_hbm.at[pl.ds(e0, chunk)], dst_v[b],
                             lsem[b])
            pltpu.async_copy(w_hbm.at[pl.ds(e0, chunk)], w_v[b], lsem[b])
            pltpu.async_copy(h_hbm.at[sv[sn]], rows[b], gsem[b])

        def wait_loads(b):
            pltpu.make_async_copy(dst_hbm.at[pl.ds(0, chunk)], dst_v[b],
                                  lsem[b]).wait()
            pltpu.make_async_copy(w_hbm.at[pl.ds(0, chunk)], w_v[b],
                                  lsem[b]).wait()

        def wait_gather(sn, b):
            pltpu.make_async_copy(h_hbm.at[sv[sn]], rows[b],
                                  gsem[b]).wait()

        def wait_scatter(b):
            pltpu.make_async_copy(rows[b], acc.at[dst_v[b]],
                                  ssem[b]).wait()

        def scale_scatter(b):
            def group(g, c2):
                b16 = g * LANES
                for i in range(LANES):
                    e = b16 + i
                    wi = plsc.load_gather(
                        w_v[b], [jnp.full((LANES,), e, jnp.int32)])
                    for k in range(dsub):
                        sl = pl.ds(k * LANES, LANES)
                        rows[b][e, sl] = rows[b][e, sl] * wi
                pltpu.async_copy(
                    rows[b].at[pl.ds(b16, LANES)],
                    acc.at[dst_v[b].at[pl.ds(b16, LANES)]],
                    ssem[b], add=True)
                return c2
            lax.fori_loop(0, n_groups, group, 0)

        def body(k, j, issue_next, issue_src_next):
            b = j % NBUF
            if issue_src_next:
                issue_src(k + SL, (j + SL) % SBUF)
            if issue_next:
                bn = (j + LOOK) % NBUF
                sn = (j + LOOK) % SBUF
                @pl.when(k >= NBUF - LOOK)
                def _w():
                    wait_scatter(bn)
                wait_src(sn)
                issue_chunk(k + LOOK, sn, bn)
            wait_gather(j % SBUF, b)
            wait_loads(b)
            scale_scatter(b)

        for j in range(SL):
            issue_src(j, j % SBUF)
        for j in range(LOOK):
            wait_src(j % SBUF)
            issue_chunk(j, j % SBUF, j % NBUF)

        UNROLL = 8
        n_main = (n_chunks - SL) // UNROLL
        def group(g, carry):
            k0 = g * UNROLL
            for j in range(UNROLL):
                body(k0 + j, j, True, True)
            return carry
        lax.fori_loop(0, n_main, group, 0)

        for k in range(n_main * UNROLL, n_chunks):
            body(k, k % UNROLL, k + LOOK < n_chunks, k + SL < n_chunks)
        for b in range(NBUF):
            wait_scatter(b)

        plsc.subcore_barrier()
        pltpu.sync_copy(acc.at[pl.ds(rb, rpt)],
                        out_hbm.at[cid, pl.ds(rb, rpt)])
        if tail:
            @pl.when(sid == NS - 1)
            def _write_tail():
                pltpu.sync_copy(acc.at[pl.ds(NS * rpt, tail)],
                                out_hbm.at[cid, pl.ds(NS * rpt, tail)])

    return spmm


def kernel(x, edge_index, edge_weight, W0, b0, W1, b1):
    n, _ = x.shape
    e = edge_index.shape[1]
    src = edge_index[0].astype(jnp.int32)
    dst = edge_index[1].astype(jnp.int32)
    ew = edge_weight.astype(jnp.float32)

    spmm = _make_spmm(n, W0.shape[1], e)

    h0 = _mm0(x, W0, 1000)
    p0 = spmm(h0, src, dst, ew)
    h1 = _mm1(p0, b0.reshape(1, -1), W1, 1000)
    p1 = spmm(h1, src, dst, ew)
    return _merge(p1, b1.reshape(1, -1), 1000)


# speedup vs baseline: 11.7369x; 1.0081x over previous
import functools

import jax
import jax.numpy as jnp
from jax import lax
from jax.experimental import pallas as pl
from jax.experimental.pallas import tpu as pltpu
from jax.experimental.pallas import tpu_sc as plsc

NC, NS, LANES = 2, 16, 16
NW = NC * NS


def _mm0_body(x_ref, w_ref, o_ref):
    o_ref[...] = jnp.dot(x_ref[...], w_ref[...],
                         preferred_element_type=jnp.float32)


def _mm1_body(p_ref, b_ref, w_ref, o_ref):
    h = p_ref[0] + p_ref[1] + b_ref[...]
    h = jnp.maximum(h, 0.0)
    o_ref[...] = jnp.dot(h, w_ref[...], preferred_element_type=jnp.float32)


def _merge_body(p_ref, b_ref, o_ref):
    o_ref[...] = p_ref[0] + p_ref[1] + b_ref[...]


def _mm0(x, w, rows_per_block):
    n, k = x.shape
    m = w.shape[1]
    return pl.pallas_call(
        _mm0_body,
        grid=(n // rows_per_block,),
        in_specs=[
            pl.BlockSpec((rows_per_block, k), lambda i: (i, 0)),
            pl.BlockSpec((k, m), lambda i: (0, 0)),
        ],
        out_specs=pl.BlockSpec((rows_per_block, m), lambda i: (i, 0)),
        out_shape=jax.ShapeDtypeStruct((n, m), jnp.float32),
    )(x, w)


def _mm1(p, b, w, rows_per_block):
    _, n, k = p.shape
    m = w.shape[1]
    return pl.pallas_call(
        _mm1_body,
        grid=(n // rows_per_block,),
        in_specs=[
            pl.BlockSpec((2, rows_per_block, k), lambda i: (0, i, 0)),
            pl.BlockSpec((1, k), lambda i: (0, 0)),
            pl.BlockSpec((k, m), lambda i: (0, 0)),
        ],
        out_specs=pl.BlockSpec((rows_per_block, m), lambda i: (i, 0)),
        out_shape=jax.ShapeDtypeStruct((n, m), jnp.float32),
    )(p, b, w)


def _merge(p, b, rows_per_block):
    _, n, k = p.shape
    return pl.pallas_call(
        _merge_body,
        grid=(n // rows_per_block,),
        in_specs=[
            pl.BlockSpec((2, rows_per_block, k), lambda i: (0, i, 0)),
            pl.BlockSpec((1, k), lambda i: (0, 0)),
        ],
        out_specs=pl.BlockSpec((rows_per_block, k), lambda i: (i, 0)),
        out_shape=jax.ShapeDtypeStruct((n, k), jnp.float32),
    )(p, b)


def _make_spmm(n_nodes, d, n_edges):
    assert n_edges % NW == 0
    epw = n_edges // NW
    chunk = 80
    assert epw % chunk == 0
    n_chunks = epw // chunk
    n_groups = chunk // LANES
    rpt = (n_nodes // NS) // 8 * 8
    tail = n_nodes - NS * rpt
    assert 0 <= tail <= rpt and tail % 8 == 0
    dsub = d // LANES

    mesh = plsc.VectorSubcoreMesh(
        core_axis_name="c", subcore_axis_name="s",
        num_cores=NC, num_subcores=NS)

    NBUF = 4
    LOOK = 2
    SBUF = 8
    SL = 4

    @functools.partial(
        pl.kernel,
        out_type=jax.ShapeDtypeStruct((NC, n_nodes, d), jnp.float32),
        mesh=mesh,
        compiler_params=pltpu.CompilerParams(needs_layout_passes=False),
        scratch_types=[
            pltpu.VMEM_SHARED((n_nodes, d), jnp.float32),
            [pltpu.VMEM((chunk,), jnp.int32)] * SBUF,
            [pltpu.VMEM((chunk,), jnp.int32)] * NBUF,
            [pltpu.VMEM((chunk,), jnp.float32)] * NBUF,
            [pltpu.VMEM((chunk, d), jnp.float32)] * NBUF,
            [pltpu.SemaphoreType.DMA] * SBUF,
            [pltpu.SemaphoreType.DMA] * NBUF,
            [pltpu.SemaphoreType.DMA] * NBUF,
            [pltpu.SemaphoreType.DMA] * NBUF,
        ],
    )
    def spmm(h_hbm, src_hbm, dst_hbm, w_hbm, out_hbm,
             acc, sv, dst_v, w_v, rows, ksem, gsem, lsem, ssem):
        cid = lax.axis_index("c")
        sid = lax.axis_index("s")
        ebase = (cid * NS + sid) * epw
        rb = sid * rpt

        for j in range(SL):
            pltpu.async_copy(src_hbm.at[pl.ds(ebase + j * chunk, chunk)],
                             sv[j % SBUF], ksem[j % SBUF])

        def zrow(r, carry):
            for k in range(dsub):
                rows[0][r, pl.ds(k * LANES, LANES)] = jnp.zeros(
                    (LANES,), jnp.float32)
            return carry
        lax.fori_loop(0, chunk, zrow, 0)

        def zcopy(j, carry):
            pltpu.sync_copy(rows[0], acc.at[pl.ds(rb + j * chunk, chunk)])
            return carry
        nzc = rpt // chunk
        lax.fori_loop(0, nzc, zcopy, 0)
        zrem = rpt - nzc * chunk
        if zrem:
            pltpu.sync_copy(rows[0].at[pl.ds(0, zrem)],
                            acc.at[pl.ds(rb + nzc * chunk, zrem)])
        if tail:
            @pl.when(sid == NS - 1)
            def _zero_tail():
                pltpu.sync_copy(rows[0].at[pl.ds(0, tail)],
                                acc.at[pl.ds(NS * rpt, tail)])
        plsc.subcore_barrier()

        def issue_src(k, bs):
            pltpu.async_copy(src_hbm.at[pl.ds(ebase + k * chunk, chunk)],
                             sv[bs], ksem[bs])

        def wait_src(bs):
            pltpu.make_async_copy(src_hbm.at[pl.ds(0, chunk)], sv[bs],
                                  ksem[bs]).wait()

        def issue_chunk(k, sn, b):
            e0 = ebase + k * chunk
            pltpu.async_copy(dst_hbm.at[pl.ds(e0, chunk)], dst_v[b],
                             lsem[b])
            pltpu.async_copy(w_hbm.at[pl.ds(e0, chunk)], w_v[b], lsem[b])
            pltpu.async_copy(h_hbm.at[sv[sn]], rows[b], gsem[b])

        def wait_loads(b):
            pltpu.make_async_copy(dst_hbm.at[pl.ds(0, chunk)], dst_v[b],
                                  lsem[b]).wait()
            pltpu.make_async_copy(w_hbm.at[pl.ds(0, chunk)], w_v[b],
                                  lsem[b]).wait()

        def wait_gather(sn, b):
            pltpu.make_async_copy(h_hbm.at[sv[sn]], rows[b],
                                  gsem[b]).wait()

        def wait_scatter(b):
            pltpu.make_async_copy(rows[b], acc.at[dst_v[b]],
                                  ssem[b]).wait()

        def scale_scatter(b):
            def group(g, c2):
                b16 = g * LANES
                for i in range(LANES):
                    e = b16 + i
                    wi = plsc.load_gather(
                        w_v[b], [jnp.full((LANES,), e, jnp.int32)])
                    for k in range(dsub):
                        sl = pl.ds(k * LANES, LANES)
                        rows[b][e, sl] = rows[b][e, sl] * wi
                pltpu.async_copy(
                    rows[b].at[pl.ds(b16, LANES)],
                    acc.at[dst_v[b].at[pl.ds(b16, LANES)]],
                    ssem[b], add=True)
                return c2
            lax.fori_loop(0, n_groups, group, 0)

        def body(k, j, issue_next, issue_src_next):
            b = j % NBUF
            if issue_src_next:
                issue_src(k + SL, (j + SL) % SBUF)
            if issue_next:
                bn = (j + LOOK) % NBUF
                sn = (j + LOOK) % SBUF
                @pl.when(k >= NBUF - LOOK)
                def _w():
                    wait_scatter(bn)
                wait_src(sn)
                issue_chunk(k + LOOK, sn, bn)
            wait_gather(j % SBUF, b)
            wait_loads(b)
            scale_scatter(b)

        for j in range(LOOK):
            wait_src(j % SBUF)
            issue_chunk(j, j % SBUF, j % NBUF)

        UNROLL = 8
        n_main = (n_chunks - SL) // UNROLL
        def group(g, carry):
            k0 = g * UNROLL
            for j in range(UNROLL):
                body(k0 + j, j, True, True)
            return carry
        lax.fori_loop(0, n_main, group, 0)

        for k in range(n_main * UNROLL, n_chunks):
            body(k, k % UNROLL, k + LOOK < n_chunks, k + SL < n_chunks)
        for b in range(NBUF):
            wait_scatter(b)

        plsc.subcore_barrier()
        pltpu.sync_copy(acc.at[pl.ds(rb, rpt)],
                        out_hbm.at[cid, pl.ds(rb, rpt)])
        if tail:
            @pl.when(sid == NS - 1)
            def _write_tail():
                pltpu.sync_copy(acc.at[pl.ds(NS * rpt, tail)],
                                out_hbm.at[cid, pl.ds(NS * rpt, tail)])

    return spmm


def kernel(x, edge_index, edge_weight, W0, b0, W1, b1):
    n, _ = x.shape
    e = edge_index.shape[1]
    src = edge_index[0].astype(jnp.int32)
    dst = edge_index[1].astype(jnp.int32)
    ew = edge_weight.astype(jnp.float32)

    spmm = _make_spmm(n, W0.shape[1], e)

    h0 = _mm0(x, W0, 1000)
    p0 = spmm(h0, src, dst, ew)
    h1 = _mm1(p0, b0.reshape(1, -1), W1, 1000)
    p1 = spmm(h1, src, dst, ew)
    return _merge(p1, b1.reshape(1, -1), 1000)
